# baseline (speedup 1.0000x reference)
import functools

import jax
import jax.numpy as jnp
from jax import lax
from jax.experimental import pallas as pl
from jax.experimental.pallas import tpu as pltpu
from jax.experimental.pallas import tpu_sc as plsc

N = 10000
E = 320000
H = 128
NTILES = 16
NW = 32
C = 128
CHUNKS = 80
E_PAD = NW * C * CHUNKS
MACROS = CHUNKS // 2
BLK = 16
DCHUNKS = 80
DE_PAD = NW * C * DCHUNKS
NBLK = DCHUNKS // BLK
ROWS_PAD = 10112
ZROWS = 632
DEG_PAD = 10240
F32 = jnp.float32


@functools.partial(
    pl.kernel,
    out_type=(jax.ShapeDtypeStruct((N,), F32),
              jax.ShapeDtypeStruct((N,), F32)),
    mesh=plsc.VectorSubcoreMesh(
        core_axis_name="c", subcore_axis_name="s", num_cores=2),
    scratch_types=dict(
        acc=pltpu.VMEM_SHARED((DEG_PAD,), F32),
        dbigA=pltpu.VMEM((BLK, C), jnp.int32),
        dbigB=pltpu.VMEM((BLK, C), jnp.int32),
        ones=pltpu.VMEM((C,), F32),
        dbuf=pltpu.VMEM((ZROWS,), F32),
        isemA=pltpu.SemaphoreType.DMA,
        isemB=pltpu.SemaphoreType.DMA,
        ssem=pltpu.SemaphoreType.DMA,
    ),
)
def _deg_kernel(dst2_hbm, zeros1_hbm, deg0_out, deg1_out,
                acc, dbigA, dbigB, ones, dbuf, isemA, isemB, ssem):
    cid = lax.axis_index("c")
    t = lax.axis_index("s")
    wid = cid * NTILES + t
    pltpu.sync_copy(zeros1_hbm.at[pl.ds(t * 640, 640)],
                    acc.at[pl.ds(t * 640, 640)])
    for j in range(C // 16):
        ones[pl.ds(j * 16, 16)] = jnp.ones((16,), F32)
    plsc.subcore_barrier()

    bufs = (dbigA, dbigB)
    sems = (isemA, isemB)

    def ifetch(b, dbig, isem):
        row0 = pl.multiple_of(wid * DCHUNKS + b * BLK, 8)
        pltpu.async_copy(dst2_hbm.at[pl.ds(row0, BLK)], dbig, isem)

    ifetch(0, bufs[0], sems[0])
    for b in range(NBLK):
        dbig, isem = bufs[b % 2], sems[b % 2]
        pltpu.make_async_copy(
            dst2_hbm.at[pl.ds(0, BLK)], dbig, isem).wait()
        if b + 1 < NBLK:
            ifetch(b + 1, bufs[(b + 1) % 2], sems[(b + 1) % 2])
        for k in range(BLK):
            pltpu.async_copy(ones, acc.at[dbig.at[k]], ssem, add=True)
        for k in range(BLK):
            pltpu.make_async_copy(ones, acc.at[dbig.at[0]], ssem).wait()

    plsc.subcore_barrier()
    off = pl.multiple_of(t * ZROWS, 8)

    @pl.when(t < NTILES - 1)
    def _():
        pltpu.sync_copy(acc.at[pl.ds(off, ZROWS)], dbuf)

        @pl.when(cid == 0)
        def _():
            pltpu.sync_copy(dbuf, deg0_out.at[pl.ds(off, ZROWS)])

        @pl.when(cid == 1)
        def _():
            pltpu.sync_copy(dbuf, deg1_out.at[pl.ds(off, ZROWS)])

    @pl.when(t == NTILES - 1)
    def _():
        off15 = pl.multiple_of((NTILES - 1) * ZROWS, 8)
        pltpu.sync_copy(acc.at[pl.ds(off15, 520)], dbuf.at[pl.ds(0, 520)])

        @pl.when(cid == 0)
        def _():
            pltpu.sync_copy(dbuf.at[pl.ds(0, 520)],
                            deg0_out.at[pl.ds(off15, 520)])

        @pl.when(cid == 1)
        def _():
            pltpu.sync_copy(dbuf.at[pl.ds(0, 520)],
                            deg1_out.at[pl.ds(off15, 520)])


@functools.partial(
    pl.kernel,
    out_type=(jax.ShapeDtypeStruct((ROWS_PAD, H), F32),
              jax.ShapeDtypeStruct((ROWS_PAD, H), F32)),
    mesh=plsc.VectorSubcoreMesh(
        core_axis_name="c", subcore_axis_name="s", num_cores=2),
    scratch_types=dict(
        acc=pltpu.VMEM_SHARED((ROWS_PAD, H), F32),
        sdA=pltpu.VMEM((4, C), jnp.int32),
        sdB=pltpu.VMEM((4, C), jnp.int32),
        rows0=pltpu.VMEM((C, H), F32),
        rows1=pltpu.VMEM((C, H), F32),
        gsem0=pltpu.SemaphoreType.DMA,
        gsem1=pltpu.SemaphoreType.DMA,
    ),
)
def _hop_kernel(w_hbm, sd3_hbm, zeros2_hbm, p0_out, p1_out,
                acc, sdA, sdB, rows0, rows1, gsem0, gsem1):
    cid = lax.axis_index("c")
    t = lax.axis_index("s")
    wid = cid * NTILES + t
    rows = (rows0, rows1)
    gsem = (gsem0, gsem1)

    pltpu.sync_copy(zeros2_hbm.at[pl.ds(0, ZROWS)],
                    acc.at[pl.ds(pl.multiple_of(t * ZROWS, 8), ZROWS)])
    plsc.subcore_barrier()

    def gwait(j):
        pltpu.make_async_copy(w_hbm.at[pl.ds(0, C)], rows[j],
                              gsem[j]).wait()

    m0 = wid * MACROS

    def macro(m, sd, sd_next, prefetch_pred):
        pltpu.async_copy(w_hbm.at[sd.at[1]], rows1, gsem1)
        gwait(0)
        pltpu.sync_copy(rows0, acc.at[sd.at[2]], add=True)

        @pl.when(prefetch_pred)
        def _():
            pltpu.sync_copy(sd3_hbm.at[m + 1], sd_next)
            pltpu.async_copy(w_hbm.at[sd_next.at[0]], rows0, gsem0)

        gwait(1)
        pltpu.sync_copy(rows1, acc.at[sd.at[3]], add=True)

    pltpu.sync_copy(sd3_hbm.at[m0], sdA)
    pltpu.async_copy(w_hbm.at[sdA.at[0]], rows0, gsem0)

    def body(j2, carry):
        m = m0 + j2 * 2
        macro(m, sdA, sdB, j2 * 2 + 1 < MACROS)
        macro(m + 1, sdB, sdA, j2 * 2 + 2 < MACROS)
        return carry

    lax.fori_loop(0, MACROS // 2, body, 0)
    plsc.subcore_barrier()

    stripe = pl.ds(pl.multiple_of(t * ZROWS, 8), ZROWS)

    @pl.when(cid == 0)
    def _():
        pltpu.sync_copy(acc.at[stripe], p0_out.at[stripe])

    @pl.when(cid == 1)
    def _():
        pltpu.sync_copy(acc.at[stripe], p1_out.at[stripe])


def _embed_body(h_ref, we_ref, be_ref, d0_ref, d1_ref,
                hcur_ref, w_ref, rs_ref, rs2_ref):
    degc = jnp.maximum(d0_ref[...] + d1_ref[...], 1.0)
    rs = lax.rsqrt(degc)
    rs_ref[...] = rs
    rs2_ref[...] = 1.0 / degc
    hcur = jnp.dot(h_ref[...], we_ref[...],
                   preferred_element_type=F32) + be_ref[...]
    hcur_ref[...] = hcur
    w_ref[...] = hcur * rs


def _embed(h, We, be, d0_col, d1_col):
    return pl.pallas_call(
        _embed_body,
        out_shape=(jax.ShapeDtypeStruct((N, H), F32),
                   jax.ShapeDtypeStruct((N, H), F32),
                   jax.ShapeDtypeStruct((N, 1), F32),
                   jax.ShapeDtypeStruct((N, 1), F32)),
    )(h, We, be, d0_col, d1_col)


def _combine_body(p0_ref, p1_ref, rs2_ref, a_ref, w_ref):
    a = p0_ref[0:N, :] + p1_ref[0:N, :]
    a_ref[...] = a
    w_ref[...] = a * rs2_ref[...]


def _combine(p0, p1, rs2_col):
    return pl.pallas_call(
        _combine_body,
        out_shape=(jax.ShapeDtypeStruct((N, H), F32),
                   jax.ShapeDtypeStruct((N, H), F32)),
    )(p0, p1, rs2_col)


def _layer_body(hin_ref, a1_ref, a2_ref, p30_ref, p31_ref, rs_ref, sn_ref,
                A_ref, Wf_ref, bf_ref, g_ref, b_ref, hout_ref, wout_ref):
    hin = hin_ref[...]
    rs = rs_ref[...]
    third = 1.0 / 3.0

    def M(k):
        return (A_ref[0, k] * Wf_ref[0] + A_ref[1, k] * Wf_ref[1]
                + A_ref[2, k] * Wf_ref[2]) * third

    hc = jnp.dot(hin, M(0), preferred_element_type=F32)
    hc = hc + jnp.dot(a1_ref[...] * rs, M(1), preferred_element_type=F32)
    hc = hc + jnp.dot(a2_ref[...] * rs, M(2), preferred_element_type=F32)
    hc = hc + jnp.dot((p30_ref[0:N, :] + p31_ref[0:N, :]) * rs, M(3),
                      preferred_element_type=F32)
    bbar = (bf_ref[0] + bf_ref[1] + bf_ref[2]) * third
    hc = (hc + bbar) * sn_ref[...]
    mu = jnp.mean(hc, axis=0, keepdims=True)
    var = jnp.mean((hc - mu) ** 2, axis=0, keepdims=True)
    hc = (hc - mu) * lax.rsqrt(var + 1e-5) * g_ref[...] + b_ref[...]
    hc = jnp.maximum(hc, 0.0)
    hout = hc + hin
    hout_ref[...] = hout
    wout_ref[...] = hout * rs


def _layer(hin, a1, a2, p30, p31, rs_col, snorm_n, A_l, Wf_l, bf_l, g_l,
           b_l):
    return pl.pallas_call(
        _layer_body,
        out_shape=(jax.ShapeDtypeStruct((N, H), F32),
                   jax.ShapeDtypeStruct((N, H), F32)),
        in_specs=[pl.BlockSpec(memory_space=pltpu.MemorySpace.VMEM)] * 7
        + [pl.BlockSpec(memory_space=pltpu.MemorySpace.SMEM)]
        + [pl.BlockSpec(memory_space=pltpu.MemorySpace.VMEM)] * 4,
        compiler_params=pltpu.CompilerParams(
            vmem_limit_bytes=63 * 1024 * 1024),
    )(hin, a1, a2, p30, p31, rs_col, snorm_n, A_l, Wf_l, bf_l, g_l, b_l)


def _readout_body(hc_ref, w1_ref, b1_ref, w2_ref, b2_ref, w3_ref, b3_ref,
                  out_ref):
    hg = jnp.mean(hc_ref[...], axis=0, keepdims=True)
    x = jnp.maximum(jnp.dot(hg, w1_ref[...],
                            preferred_element_type=F32) + b1_ref[...], 0.0)
    x = jnp.maximum(jnp.dot(x, w2_ref[...],
                            preferred_element_type=F32) + b2_ref[...], 0.0)
    out_ref[...] = jnp.dot(x, w3_ref[...],
                           preferred_element_type=F32) + b3_ref[...]


def _readout(hc, W1, b1, W2, b2, W3, b3):
    return pl.pallas_call(
        _readout_body,
        out_shape=jax.ShapeDtypeStruct((1, 10), F32),
    )(hc, W1, b1, W2, b2, W3, b3)


def kernel(h, edge_index, e, snorm_n, snorm_e, W_embed, b_embed, A_coef,
           Wf, bf, gamma, beta, W1, b1, W2, b2, W3, b3):
    src = edge_index[0]
    dst = edge_index[1]
    i32 = jnp.int32
    srcF = jnp.concatenate([src, jnp.zeros((E_PAD - E,), i32)])
    dstF = jnp.concatenate([dst, jnp.full((E_PAD - E,), N, i32)])
    sd3 = jnp.concatenate(
        [srcF.reshape(NW * MACROS, 2, C), dstF.reshape(NW * MACROS, 2, C)],
        axis=1)
    dst2 = jnp.concatenate(
        [dst, jnp.full((DE_PAD - E,), N, i32)]).reshape(NW * DCHUNKS, C)
    zeros1 = jnp.zeros((DEG_PAD,), F32)
    zeros2 = jnp.zeros((ZROWS, H), F32)

    d0, d1 = _deg_kernel(dst2, zeros1)
    hcur, w, rs_col, rs2_col = _embed(
        h, W_embed, b_embed.reshape(1, H), d0[:, None], d1[:, None])
    for l in range(4):
        hin = hcur
        a_list = []
        for _ in range(2):
            p0, p1 = _hop_kernel(w, sd3, zeros2)
            a, w = _combine(p0, p1, rs2_col)
            a_list.append(a)
        p30, p31 = _hop_kernel(w, sd3, zeros2)
        hcur, w = _layer(hin, a_list[0], a_list[1], p30, p31, rs_col,
                         snorm_n, A_coef[l], Wf[l], bf[l].reshape(3, 1, H),
                         gamma[l].reshape(1, H), beta[l].reshape(1, H))
    out = _readout(hcur, W1, b1.reshape(1, -1), W2, b2.reshape(1, -1),
                   W3, b3.reshape(1, -1))
    return out


# speedup vs baseline: 1.8329x; 1.8329x over previous
import functools

import jax
import jax.numpy as jnp
from jax import lax
from jax.experimental import pallas as pl
from jax.experimental.pallas import tpu as pltpu
from jax.experimental.pallas import tpu_sc as plsc

N = 10000
E = 320000
H = 128
NTILES = 16
NW = 32
C = 128
CHUNKS = 79
E_PAD = NW * C * CHUNKS
BLK = 16
DCHUNKS = 80
DE_PAD = NW * C * DCHUNKS
NBLK = DCHUNKS // BLK
ROWS_PAD = 10112
ZROWS = 632
DEG_PAD = 10240
F32 = jnp.float32


@functools.partial(
    pl.kernel,
    out_type=(jax.ShapeDtypeStruct((N,), F32),
              jax.ShapeDtypeStruct((N,), F32)),
    mesh=plsc.VectorSubcoreMesh(
        core_axis_name="c", subcore_axis_name="s", num_cores=2),
    scratch_types=dict(
        acc=pltpu.VMEM_SHARED((DEG_PAD,), F32),
        dbigA=pltpu.VMEM((BLK, C), jnp.int32),
        dbigB=pltpu.VMEM((BLK, C), jnp.int32),
        ones=pltpu.VMEM((C,), F32),
        dbuf=pltpu.VMEM((ZROWS,), F32),
        isemA=pltpu.SemaphoreType.DMA,
        isemB=pltpu.SemaphoreType.DMA,
        ssem=pltpu.SemaphoreType.DMA,
    ),
)
def _deg_kernel(dst2_hbm, zeros1_hbm, deg0_out, deg1_out,
                acc, dbigA, dbigB, ones, dbuf, isemA, isemB, ssem):
    cid = lax.axis_index("c")
    t = lax.axis_index("s")
    wid = cid * NTILES + t
    pltpu.sync_copy(zeros1_hbm.at[pl.ds(t * 640, 640)],
                    acc.at[pl.ds(t * 640, 640)])
    for j in range(C // 16):
        ones[pl.ds(j * 16, 16)] = jnp.ones((16,), F32)
    plsc.subcore_barrier()

    bufs = (dbigA, dbigB)
    sems = (isemA, isemB)

    def ifetch(b, dbig, isem):
        row0 = pl.multiple_of(wid * DCHUNKS + b * BLK, 8)
        pltpu.async_copy(dst2_hbm.at[pl.ds(row0, BLK)], dbig, isem)

    ifetch(0, bufs[0], sems[0])
    for b in range(NBLK):
        dbig, isem = bufs[b % 2], sems[b % 2]
        pltpu.make_async_copy(
            dst2_hbm.at[pl.ds(0, BLK)], dbig, isem).wait()
        if b + 1 < NBLK:
            ifetch(b + 1, bufs[(b + 1) % 2], sems[(b + 1) % 2])
        for k in range(BLK):
            pltpu.async_copy(ones, acc.at[dbig.at[k]], ssem, add=True)
        for k in range(BLK):
            pltpu.make_async_copy(ones, acc.at[dbig.at[0]], ssem).wait()

    plsc.subcore_barrier()
    off = pl.multiple_of(t * ZROWS, 8)

    @pl.when(t < NTILES - 1)
    def _():
        pltpu.sync_copy(acc.at[pl.ds(off, ZROWS)], dbuf)

        @pl.when(cid == 0)
        def _():
            pltpu.sync_copy(dbuf, deg0_out.at[pl.ds(off, ZROWS)])

        @pl.when(cid == 1)
        def _():
            pltpu.sync_copy(dbuf, deg1_out.at[pl.ds(off, ZROWS)])

    @pl.when(t == NTILES - 1)
    def _():
        off15 = pl.multiple_of((NTILES - 1) * ZROWS, 8)
        pltpu.sync_copy(acc.at[pl.ds(off15, 520)], dbuf.at[pl.ds(0, 520)])

        @pl.when(cid == 0)
        def _():
            pltpu.sync_copy(dbuf.at[pl.ds(0, 520)],
                            deg0_out.at[pl.ds(off15, 520)])

        @pl.when(cid == 1)
        def _():
            pltpu.sync_copy(dbuf.at[pl.ds(0, 520)],
                            deg1_out.at[pl.ds(off15, 520)])


@functools.partial(
    pl.kernel,
    out_type=(jax.ShapeDtypeStruct((ROWS_PAD, H), F32),
              jax.ShapeDtypeStruct((ROWS_PAD, H), F32)),
    mesh=plsc.VectorSubcoreMesh(
        core_axis_name="c", subcore_axis_name="s", num_cores=2),
    scratch_types=dict(
        acc=pltpu.VMEM_SHARED((ROWS_PAD, H), F32),
        sidx0=pltpu.VMEM((C,), jnp.int32),
        sidx1=pltpu.VMEM((C,), jnp.int32),
        didx0=pltpu.VMEM((C,), jnp.int32),
        didx1=pltpu.VMEM((C,), jnp.int32),
        rows0=pltpu.VMEM((C, H), F32),
        rows1=pltpu.VMEM((C, H), F32),
        gsem0=pltpu.SemaphoreType.DMA,
        gsem1=pltpu.SemaphoreType.DMA,
    ),
)
def _hop_kernel(w_hbm, src_hbm, dst_hbm, zeros2_hbm, p0_out, p1_out,
                acc, sidx0, sidx1, didx0, didx1, rows0, rows1,
                gsem0, gsem1):
    cid = lax.axis_index("c")
    t = lax.axis_index("s")
    wid = cid * NTILES + t
    pltpu.sync_copy(zeros2_hbm.at[pl.ds(0, ZROWS)],
                    acc.at[pl.ds(pl.multiple_of(t * ZROWS, 8), ZROWS)])
    plsc.subcore_barrier()

    def fetch(i, sidx, didx, rows, gsem):
        base = pl.multiple_of((wid * CHUNKS + i) * C, C)
        pltpu.sync_copy(src_hbm.at[pl.ds(base, C)], sidx)
        pltpu.sync_copy(dst_hbm.at[pl.ds(base, C)], didx)
        pltpu.async_copy(w_hbm.at[sidx], rows, gsem)

    def drain(didx, rows, gsem):
        pltpu.make_async_copy(w_hbm.at[pl.ds(0, C)], rows, gsem).wait()
        pltpu.sync_copy(rows, acc.at[didx], add=True)

    fetch(0, sidx0, didx0, rows0, gsem0)

    def body2(i2, carry):
        i = i2 * 2

        @pl.when(i + 1 < CHUNKS)
        def _():
            fetch(i + 1, sidx1, didx1, rows1, gsem1)

        drain(didx0, rows0, gsem0)

        @pl.when(i + 2 < CHUNKS)
        def _():
            fetch(i + 2, sidx0, didx0, rows0, gsem0)

        @pl.when(i + 1 < CHUNKS)
        def _():
            drain(didx1, rows1, gsem1)

        return carry

    lax.fori_loop(0, (CHUNKS + 1) // 2, body2, 0)
    plsc.subcore_barrier()

    stripe = pl.ds(pl.multiple_of(t * ZROWS, 8), ZROWS)

    @pl.when(cid == 0)
    def _():
        pltpu.sync_copy(acc.at[stripe], p0_out.at[stripe])

    @pl.when(cid == 1)
    def _():
        pltpu.sync_copy(acc.at[stripe], p1_out.at[stripe])


def _embed_body(h_ref, we_ref, be_ref, d0_ref, d1_ref,
                hcur_ref, w_ref, rs_ref, rs2_ref):
    degc = jnp.maximum(d0_ref[...] + d1_ref[...], 1.0)
    rs = lax.rsqrt(degc)
    rs_ref[...] = rs
    rs2_ref[...] = 1.0 / degc
    hcur = jnp.dot(h_ref[...], we_ref[...],
                   preferred_element_type=F32) + be_ref[...]
    hcur_ref[...] = hcur
    w_ref[...] = hcur * rs


def _embed(h, We, be, d0_col, d1_col):
    return pl.pallas_call(
        _embed_body,
        out_shape=(jax.ShapeDtypeStruct((N, H), F32),
                   jax.ShapeDtypeStruct((N, H), F32),
                   jax.ShapeDtypeStruct((N, 1), F32),
                   jax.ShapeDtypeStruct((N, 1), F32)),
    )(h, We, be, d0_col, d1_col)


def _combine_body(p0_ref, p1_ref, rs2_ref, a_ref, w_ref):
    a = p0_ref[0:N, :] + p1_ref[0:N, :]
    a_ref[...] = a
    w_ref[...] = a * rs2_ref[...]


def _combine(p0, p1, rs2_col):
    return pl.pallas_call(
        _combine_body,
        out_shape=(jax.ShapeDtypeStruct((N, H), F32),
                   jax.ShapeDtypeStruct((N, H), F32)),
    )(p0, p1, rs2_col)


def _layer_body(hin_ref, a1_ref, a2_ref, p30_ref, p31_ref, rs_ref, sn_ref,
                A_ref, Wf_ref, bf_ref, g_ref, b_ref, hout_ref, wout_ref):
    hin = hin_ref[...]
    rs = rs_ref[...]
    third = 1.0 / 3.0

    def M(k):
        return (A_ref[0, k] * Wf_ref[0] + A_ref[1, k] * Wf_ref[1]
                + A_ref[2, k] * Wf_ref[2]) * third

    hc = jnp.dot(hin, M(0), preferred_element_type=F32)
    hc = hc + jnp.dot(a1_ref[...] * rs, M(1), preferred_element_type=F32)
    hc = hc + jnp.dot(a2_ref[...] * rs, M(2), preferred_element_type=F32)
    hc = hc + jnp.dot((p30_ref[0:N, :] + p31_ref[0:N, :]) * rs, M(3),
                      preferred_element_type=F32)
    bbar = (bf_ref[0] + bf_ref[1] + bf_ref[2]) * third
    hc = (hc + bbar) * sn_ref[...]
    mu = jnp.mean(hc, axis=0, keepdims=True)
    var = jnp.mean((hc - mu) ** 2, axis=0, keepdims=True)
    hc = (hc - mu) * lax.rsqrt(var + 1e-5) * g_ref[...] + b_ref[...]
    hc = jnp.maximum(hc, 0.0)
    hout = hc + hin
    hout_ref[...] = hout
    wout_ref[...] = hout * rs


def _layer(hin, a1, a2, p30, p31, rs_col, snorm_n, A_l, Wf_l, bf_l, g_l,
           b_l):
    return pl.pallas_call(
        _layer_body,
        out_shape=(jax.ShapeDtypeStruct((N, H), F32),
                   jax.ShapeDtypeStruct((N, H), F32)),
        in_specs=[pl.BlockSpec(memory_space=pltpu.MemorySpace.VMEM)] * 7
        + [pl.BlockSpec(memory_space=pltpu.MemorySpace.SMEM)]
        + [pl.BlockSpec(memory_space=pltpu.MemorySpace.VMEM)] * 4,
        compiler_params=pltpu.CompilerParams(
            vmem_limit_bytes=63 * 1024 * 1024),
    )(hin, a1, a2, p30, p31, rs_col, snorm_n, A_l, Wf_l, bf_l, g_l, b_l)


def _readout_body(hc_ref, w1_ref, b1_ref, w2_ref, b2_ref, w3_ref, b3_ref,
                  out_ref):
    hg = jnp.mean(hc_ref[...], axis=0, keepdims=True)
    x = jnp.maximum(jnp.dot(hg, w1_ref[...],
                            preferred_element_type=F32) + b1_ref[...], 0.0)
    x = jnp.maximum(jnp.dot(x, w2_ref[...],
                            preferred_element_type=F32) + b2_ref[...], 0.0)
    out_ref[...] = jnp.dot(x, w3_ref[...],
                           preferred_element_type=F32) + b3_ref[...]


def _readout(hc, W1, b1, W2, b2, W3, b3):
    return pl.pallas_call(
        _readout_body,
        out_shape=jax.ShapeDtypeStruct((1, 10), F32),
    )(hc, W1, b1, W2, b2, W3, b3)


def kernel(h, edge_index, e, snorm_n, snorm_e, W_embed, b_embed, A_coef,
           Wf, bf, gamma, beta, W1, b1, W2, b2, W3, b3):
    src = edge_index[0]
    dst = edge_index[1]
    i32 = jnp.int32
    srcF = jnp.concatenate([src, jnp.zeros((E_PAD - E,), i32)])
    dstF = jnp.concatenate([dst, jnp.full((E_PAD - E,), N, i32)])
    dst2 = jnp.concatenate(
        [dst, jnp.full((DE_PAD - E,), N, i32)]).reshape(NW * DCHUNKS, C)
    zeros1 = jnp.zeros((DEG_PAD,), F32)
    zeros2 = jnp.zeros((ZROWS, H), F32)

    d0, d1 = _deg_kernel(dst2, zeros1)
    hcur, w, rs_col, rs2_col = _embed(
        h, W_embed, b_embed.reshape(1, H), d0[:, None], d1[:, None])
    for l in range(4):
        hin = hcur
        a_list = []
        for _ in range(2):
            p0, p1 = _hop_kernel(w, srcF, dstF, zeros2)
            a, w = _combine(p0, p1, rs2_col)
            a_list.append(a)
        p30, p31 = _hop_kernel(w, srcF, dstF, zeros2)
        hcur, w = _layer(hin, a_list[0], a_list[1], p30, p31, rs_col,
                         snorm_n, A_coef[l], Wf[l], bf[l].reshape(3, 1, H),
                         gamma[l].reshape(1, H), beta[l].reshape(1, H))
    out = _readout(hcur, W1, b1.reshape(1, -1), W2, b2.reshape(1, -1),
                   W3, b3.reshape(1, -1))
    return out
